# Initial kernel scaffold; baseline (speedup 1.0000x reference)
import jax
import jax.numpy as jnp
from jax.experimental import pallas as pl


def kernel(query_feat, context_feats, offset_w1, offset_b1, offset_w2, offset_b2, attn_w1, attn_b1, attn_w2, attn_b2, v_w, v_b, out_w, out_b):
    raise NotImplementedError("write your pallas kernel here")


# speedup vs baseline: 1.9133x; 1.9133x over previous
import jax
import jax.numpy as jnp
from jax import lax
from jax.experimental import pallas as pl

B_, C_, H_, W_ = 1, 96, 64, 64
NH_, K_, T_ = 4, 9, 7
HD_ = C_ // NH_
MAXOFF = 32.0
P_ = H_ * W_
TILE = 256
NTILES = P_ // TILE


def _erf(x):
    a1, a2, a3, a4, a5 = 0.254829592, -0.284496736, 1.421413741, -1.453152027, 1.061405429
    p = 0.3275911
    s = jnp.sign(x)
    ax = jnp.abs(x)
    t = 1.0 / (1.0 + p * ax)
    y = 1.0 - (((((a5 * t + a4) * t) + a3) * t + a2) * t + a1) * t * jnp.exp(-ax * ax)
    return s * y


def _gelu(x):
    return 0.5 * x * (1.0 + _erf(x * 0.7071067811865476))


def _dense_body(qpad_ref, ow1_ref, ob1_ref, ow2_ref, ob2_ref,
                aw1_ref, ab1_ref, aw2_ref, ab2_ref, off_ref, aw_ref):
    def conv3(w_ref, b_ref):
        acc = jnp.zeros((P_, C_), jnp.float32)
        for j in range(9):
            dy, dx = j // 3, j % 3
            xs = qpad_ref[dy:dy + H_, dx:dx + W_, :].reshape(P_, C_)
            acc += jnp.dot(xs, w_ref[j], preferred_element_type=jnp.float32)
        return acc + b_ref[...]

    hid_o = _gelu(conv3(ow1_ref, ob1_ref))
    off = jnp.dot(hid_o, ow2_ref[...], preferred_element_type=jnp.float32) + ob2_ref[...]
    off_ref[...] = jnp.tanh(off) * MAXOFF

    hid_a = _gelu(conv3(aw1_ref, ab1_ref))
    al = jnp.dot(hid_a, aw2_ref[...], preferred_element_type=jnp.float32) + ab2_ref[...]
    cols = []
    for h in range(NH_):
        sl = al[:, h * (T_ * K_):(h + 1) * (T_ * K_)]
        m = jnp.max(sl, axis=1, keepdims=True)
        e = jnp.exp(sl - m)
        cols.append(e / jnp.sum(e, axis=1, keepdims=True))
    aw_ref[...] = jnp.concatenate(cols, axis=1)


def _vproj_body(x_ref, w_ref, b_ref, o_ref):
    o_ref[...] = jnp.dot(x_ref[...], w_ref[...], preferred_element_type=jnp.float32) + b_ref[...]


def _gather_body(vimg_ref, offs_ref, aws_ref, qf_ref, ow_ref, ob_ref,
                 acc_ref, out_ref):
    t = pl.program_id(1)
    i = pl.program_id(0)

    p_idx = lax.broadcasted_iota(jnp.int32, (TILE, 1), 0) + i * TILE
    px = (p_idx % W_).astype(jnp.float32)
    py = (p_idx // W_).astype(jnp.float32)

    iota_y = lax.broadcasted_iota(jnp.int32, (TILE, H_), 1)
    iota_xm = lax.broadcasted_iota(jnp.int32, (TILE, HD_ * W_), 1) % W_

    jj = lax.broadcasted_iota(jnp.int32, (HD_ * W_, HD_), 0) // W_
    cc = lax.broadcasted_iota(jnp.int32, (HD_ * W_, HD_), 1)
    summat = (jj == cc).astype(jnp.float32)

    offs = offs_ref[0]
    aws = aws_ref[0]

    heads = []
    for h in range(NH_):
        vh = vimg_ref[0, h]
        acc = jnp.zeros((TILE, HD_), jnp.float32)
        for k in range(K_):
            c = (h * K_ + k) * 2
            ox = offs[:, c:c + 1]
            oy = offs[:, c + 1:c + 2]
            a_w = aws[:, h * K_ + k:h * K_ + k + 1]
            sx = px + ox
            sy = py + oy
            x0 = jnp.floor(sx)
            fx = sx - x0
            y0 = jnp.floor(sy)
            fy = sy - y0
            x0i = x0.astype(jnp.int32)
            y0i = y0.astype(jnp.int32)
            rowsel = a_w * (jnp.where(iota_y == y0i, 1.0 - fy, 0.0)
                            + jnp.where(iota_y == y0i + 1, fy, 0.0))
            r = jnp.dot(rowsel, vh, preferred_element_type=jnp.float32)
            colmask = (jnp.where(iota_xm == x0i, 1.0 - fx, 0.0)
                       + jnp.where(iota_xm == x0i + 1, fx, 0.0))
            acc += jnp.dot(r * colmask, summat, preferred_element_type=jnp.float32)
        heads.append(acc)
    res = jnp.concatenate(heads, axis=1)

    @pl.when(t == 0)
    def _():
        acc_ref[...] = jnp.zeros_like(acc_ref)

    acc_ref[...] += res

    @pl.when(t == T_ - 1)
    def _():
        out_ref[...] = (jnp.dot(acc_ref[...], ow_ref[...],
                                preferred_element_type=jnp.float32)
                        + ob_ref[...] + qf_ref[...])


def kernel(query_feat, context_feats, offset_w1, offset_b1, offset_w2, offset_b2,
           attn_w1, attn_b1, attn_w2, attn_b2, v_w, v_b, out_w, out_b):
    f32 = jnp.float32
    qf_hwc = query_feat[0].transpose(1, 2, 0)
    qpad = jnp.pad(qf_hwc, ((1, 1), (1, 1), (0, 0)))
    qf_flat = qf_hwc.reshape(P_, C_)

    ow1 = offset_w1.transpose(2, 3, 1, 0).reshape(9, C_, C_)
    aw1 = attn_w1.transpose(2, 3, 1, 0).reshape(9, C_, C_)
    ow2 = offset_w2[:, :, 0, 0].T
    aw2 = attn_w2[:, :, 0, 0].T

    off_flat, aw_flat = pl.pallas_call(
        _dense_body,
        out_shape=(jax.ShapeDtypeStruct((P_, NH_ * K_ * 2 * T_), f32),
                   jax.ShapeDtypeStruct((P_, NH_ * K_ * T_), f32)),
    )(qpad, ow1, offset_b1.reshape(1, C_), ow2, offset_b2.reshape(1, -1),
      aw1, attn_b1.reshape(1, C_), aw2, attn_b2.reshape(1, -1))

    offs_r = off_flat.reshape(P_, NH_, T_, K_, 2).transpose(2, 0, 1, 3, 4).reshape(T_, P_, NH_ * K_ * 2)
    aws_r = aw_flat.reshape(P_, NH_, T_, K_).transpose(2, 0, 1, 3).reshape(T_, P_, NH_ * K_)

    ctx_flat = context_feats[:, 0].transpose(0, 2, 3, 1).reshape(T_ * P_, C_)
    v_all = pl.pallas_call(
        _vproj_body,
        out_shape=jax.ShapeDtypeStruct((T_ * P_, C_), f32),
    )(ctx_flat, v_w[:, :, 0, 0].T, v_b.reshape(1, C_))

    vimg = (v_all.reshape(T_, H_, W_, NH_, HD_)
            .transpose(0, 3, 1, 4, 2)
            .reshape(T_, NH_, H_, HD_ * W_))

    out_flat = pl.pallas_call(
        _gather_body,
        grid=(NTILES, T_),
        in_specs=[
            pl.BlockSpec((1, NH_, H_, HD_ * W_), lambda i, t: (t, 0, 0, 0)),
            pl.BlockSpec((1, TILE, NH_ * K_ * 2), lambda i, t: (t, i, 0)),
            pl.BlockSpec((1, TILE, NH_ * K_), lambda i, t: (t, i, 0)),
            pl.BlockSpec((TILE, C_), lambda i, t: (i, 0)),
            pl.BlockSpec((C_, C_), lambda i, t: (0, 0)),
            pl.BlockSpec((1, C_), lambda i, t: (0, 0)),
        ],
        out_specs=(pl.BlockSpec((TILE, C_), lambda i, t: (i, 0)),
                   pl.BlockSpec((TILE, C_), lambda i, t: (i, 0))),
        out_shape=(jax.ShapeDtypeStruct((P_, C_), f32),
                   jax.ShapeDtypeStruct((P_, C_), f32)),
    )(vimg, offs_r, aws_r, qf_flat, out_w[:, :, 0, 0].T, out_b.reshape(1, C_))[1]

    return out_flat.reshape(H_, W_, C_).transpose(2, 0, 1)[None]
